# baseline (speedup 1.0000x reference)
import jax
import jax.numpy as jnp
from jax import lax
from jax.experimental import pallas as pl
from jax.experimental.pallas import tpu as pltpu
from jax.experimental.pallas import tpu_sc as plsc

VOCAB = 1000000
EMBED_DIM = 64
HIST = 20
BATCH = 16384

NUM_CORES = 2
NUM_SUBCORES = 16
NW = NUM_CORES * NUM_SUBCORES
B_PER_W = BATCH // NW
CB = 128
N_CHUNKS = B_PER_W // CB
VECS_PER_ROW = EMBED_DIM // 16


def _sc_body(words_hbm, table_hbm, out_hbm, idx_v, acc_v, buf_v, sem_i, sem_g):
    cid = lax.axis_index("c")
    sid = lax.axis_index("s")
    wid = sid * NUM_CORES + cid
    base = wid * B_PER_W

    def chunk_body(cb, _):
        row0 = base + cb * CB
        pltpu.async_copy(words_hbm.at[:, pl.ds(row0, CB)], idx_v, sem_i).wait()

        pltpu.async_copy(table_hbm.at[idx_v.at[0]], acc_v, sem_g).wait()

        for h in range(1, HIST):
            pltpu.async_copy(table_hbm.at[idx_v.at[h]], buf_v, sem_g).wait()

            def mul_body(r, _):
                for c in range(VECS_PER_ROW):
                    s = pl.ds(c * 16, 16)
                    acc_v[r, s] = acc_v[r, s] * buf_v[r, s]
                return ()

            lax.fori_loop(0, CB, mul_body, (), unroll=4)

        pltpu.async_copy(acc_v, out_hbm.at[pl.ds(row0, CB)], sem_i).wait()
        return ()

    lax.fori_loop(0, N_CHUNKS, chunk_body, ())


@jax.jit
def _embed_prod(words, noun_matrix):
    mesh = plsc.VectorSubcoreMesh(
        core_axis_name="c", subcore_axis_name="s",
        num_cores=NUM_CORES, num_subcores=NUM_SUBCORES)
    return pl.kernel(
        _sc_body,
        out_type=jax.ShapeDtypeStruct((BATCH, EMBED_DIM), jnp.float32),
        mesh=mesh,
        compiler_params=pltpu.CompilerParams(use_tc_tiling_on_sc=False),
        scratch_types=[
            pltpu.VMEM((HIST, CB), jnp.int32),
            pltpu.VMEM((CB, EMBED_DIM), jnp.float32),
            pltpu.VMEM((CB, EMBED_DIM), jnp.float32),
            pltpu.SemaphoreType.DMA,
            pltpu.SemaphoreType.DMA,
        ],
    )(words, noun_matrix)


def kernel(words, noun_matrix):
    return _embed_prod(words, noun_matrix)


# speedup vs baseline: 1.3277x; 1.3277x over previous
import jax
import jax.numpy as jnp
from jax import lax
from jax.experimental import pallas as pl
from jax.experimental.pallas import tpu as pltpu
from jax.experimental.pallas import tpu_sc as plsc

VOCAB = 1000000
EMBED_DIM = 64
HIST = 20
BATCH = 16384

NUM_CORES = 2
NUM_SUBCORES = 16
NW = NUM_CORES * NUM_SUBCORES
B_PER_W = BATCH // NW
CB = 32
N_CHUNKS = B_PER_W // CB
VECS_PER_ROW = EMBED_DIM // 16


def _sc_body(words_hbm, table_hbm, out_hbm, idx_v, rows_v, outb_v,
             sem_i, sem_g0, sem_g1, sem_o):
    cid = lax.axis_index("c")
    sid = lax.axis_index("s")
    wid = sid * NUM_CORES + cid
    base = wid * B_PER_W
    sem_g = (sem_g0, sem_g1)

    idx_copies = [
        pltpu.async_copy(words_hbm.at[:, pl.ds(base + cb * CB, CB)],
                         idx_v.at[cb], sem_i)
        for cb in range(N_CHUNKS)
    ]
    for c in idx_copies:
        c.wait()

    def fire_gathers(cb):
        p = cb % 2
        return [
            pltpu.async_copy(table_hbm.at[idx_v.at[cb, h]],
                             rows_v.at[p, h], sem_g[p])
            for h in range(HIST)
        ]

    def compute(cb):
        p = cb % 2

        def row_body(r, _):
            for c in range(VECS_PER_ROW):
                s = pl.ds(c * 16, 16)
                vals = [rows_v[p, h, r, s] for h in range(HIST)]
                while len(vals) > 1:
                    nxt = [vals[i] * vals[i + 1]
                           for i in range(0, len(vals) - 1, 2)]
                    if len(vals) % 2:
                        nxt.append(vals[-1])
                    vals = nxt
                outb_v[p, r, s] = vals[0]
            return ()

        lax.fori_loop(0, CB, row_body, ())

    store_copies = [None, None]

    pending = fire_gathers(0)
    for cb in range(N_CHUNKS):
        p = cb % 2
        nxt = fire_gathers(cb + 1) if cb + 1 < N_CHUNKS else []
        for c in pending:
            c.wait()
        if store_copies[p] is not None:
            store_copies[p].wait()
        compute(cb)
        store_copies[p] = pltpu.async_copy(
            outb_v.at[p], out_hbm.at[pl.ds(base + cb * CB, CB)], sem_o)
        pending = nxt

    for sc in store_copies:
        if sc is not None:
            sc.wait()


@jax.jit
def _embed_prod(words, noun_matrix):
    mesh = plsc.VectorSubcoreMesh(
        core_axis_name="c", subcore_axis_name="s",
        num_cores=NUM_CORES, num_subcores=NUM_SUBCORES)
    return pl.kernel(
        _sc_body,
        out_type=jax.ShapeDtypeStruct((BATCH, EMBED_DIM), jnp.float32),
        mesh=mesh,
        compiler_params=pltpu.CompilerParams(use_tc_tiling_on_sc=False),
        scratch_types=[
            pltpu.VMEM((N_CHUNKS, HIST, CB), jnp.int32),
            pltpu.VMEM((2, HIST, CB, EMBED_DIM), jnp.float32),
            pltpu.VMEM((2, CB, EMBED_DIM), jnp.float32),
            pltpu.SemaphoreType.DMA,
            pltpu.SemaphoreType.DMA,
            pltpu.SemaphoreType.DMA,
            pltpu.SemaphoreType.DMA,
        ],
    )(words, noun_matrix)


def kernel(words, noun_matrix):
    return _embed_prod(words, noun_matrix)
